# Initial kernel scaffold; baseline (speedup 1.0000x reference)
import jax
import jax.numpy as jnp
from jax.experimental import pallas as pl


def kernel(x, edge_index, edge_attr, batch, params):
    raise NotImplementedError("write your pallas kernel here")


# speedup vs baseline: 3.5811x; 3.5811x over previous
import functools

import jax
import jax.numpy as jnp
from jax import lax
from jax.experimental import pallas as pl
from jax.experimental.pallas import tpu as pltpu
from jax.experimental.pallas import tpu_sc as plsc

N = 10000
E = 640000
HID = 128
NUM_LAYERS = 4
EDGE_DIM = 4
NUM_GRAPHS = 64
LN_EPS = 1e-5

NC = 2
NS = 16
LANES = 16
NW = NC * NS

EB = 128
NBLK = E // EB
N_PAD = 10112
ROWS_PER_SUB = N_PAD // NS
ZR = 158
NCHUNK = HID // LANES

RB = 1000
NRB = N // RB


def _sc_msg_body(ht_hbm, src_hbm, dst_hbm, attr_hbm, wp_hbm, out_hbm,
                 agg_sh, src_v, dst_v, attr_v, rows_v, wp_v):
    core = lax.axis_index("c")
    sub = lax.axis_index("s")
    wid = sub * NC + core

    zvec = jnp.zeros((LANES,), jnp.float32)

    def zero_row(r, carry):
        for c in range(NCHUNK):
            rows_v[r, pl.ds(c * LANES, LANES)] = zvec
        return carry

    lax.fori_loop(0, EB, zero_row, 0)
    for j in range(ROWS_PER_SUB // EB):
        pltpu.sync_copy(
            rows_v,
            agg_sh.at[pl.ds(sub * ROWS_PER_SUB + j * EB, EB)])
    rem = ROWS_PER_SUB - (ROWS_PER_SUB // EB) * EB
    if rem:
        pltpu.sync_copy(
            rows_v.at[pl.ds(0, rem)],
            agg_sh.at[pl.ds(sub * ROWS_PER_SUB + (ROWS_PER_SUB // EB) * EB,
                            rem)])

    pltpu.sync_copy(wp_hbm, wp_v)
    wvec = [[wp_v[k, pl.ds(c * LANES, LANES)] for c in range(NCHUNK)]
            for k in range(EDGE_DIM)]

    plsc.subcore_barrier()

    nblk_w = (NBLK // NW) + jnp.where(wid < (NBLK % NW), 1, 0)

    def do_block(i, carry):
        off = (wid + i * NW) * EB
        pltpu.sync_copy(src_hbm.at[pl.ds(off, EB)], src_v)
        pltpu.sync_copy(dst_hbm.at[pl.ds(off, EB)], dst_v)
        pltpu.sync_copy(attr_hbm.at[pl.ds(off, EB)], attr_v)
        pltpu.sync_copy(ht_hbm.at[src_v], rows_v)

        def do_edge(e, ecarry):
            a = [plsc.load_gather(
                    attr_v,
                    [jnp.full((LANES,), e, jnp.int32),
                     jnp.full((LANES,), k, jnp.int32)])
                 for k in range(EDGE_DIM)]
            for c in range(NCHUNK):
                acc = rows_v[e, pl.ds(c * LANES, LANES)]
                for k in range(EDGE_DIM):
                    acc = acc + a[k] * wvec[k][c]
                rows_v[e, pl.ds(c * LANES, LANES)] = jnp.maximum(acc, 0.0)
            return ecarry

        lax.fori_loop(0, EB, do_edge, 0)
        pltpu.sync_copy(rows_v, agg_sh.at[dst_v], add=True)
        return carry

    lax.fori_loop(0, nblk_w, do_block, 0)

    plsc.subcore_barrier()

    pltpu.sync_copy(
        agg_sh.at[pl.ds(sub * ROWS_PER_SUB, ROWS_PER_SUB)],
        out_hbm.at[core, pl.ds(sub * ROWS_PER_SUB, ROWS_PER_SUB)])


_sc_layer = pl.kernel(
    _sc_msg_body,
    out_type=jax.ShapeDtypeStruct((NC, N_PAD, HID), jnp.float32),
    mesh=plsc.VectorSubcoreMesh(core_axis_name="c", subcore_axis_name="s"),
    scratch_types=[
        pltpu.VMEM_SHARED((N_PAD, HID), jnp.float32),
        pltpu.VMEM((EB,), jnp.int32),
        pltpu.VMEM((EB,), jnp.int32),
        pltpu.VMEM((EB, EDGE_DIM), jnp.float32),
        pltpu.VMEM((EB, HID), jnp.float32),
        pltpu.VMEM((EDGE_DIM, HID), jnp.float32),
    ],
    compiler_params=pltpu.CompilerParams(needs_layout_passes=False),
)


def _prep_body(edge_W_ref, edge_b_ref, lin_Ws_ref, lin_bs_ref,
               wp_ref, bp_ref):
    ew = edge_W_ref[...]
    eb = edge_b_ref[...]
    for l in range(NUM_LAYERS):
        lw = lin_Ws_ref[l]
        wp_ref[l] = jnp.dot(ew, lw, preferred_element_type=jnp.float32)
        bp_ref[pl.ds(l, 1), :] = (
            jnp.dot(eb, lw, preferred_element_type=jnp.float32)
            + lin_bs_ref[pl.ds(l, 1), :])


def _prep(edge_W, edge_b, lin_Ws, lin_bs):
    return pl.pallas_call(
        _prep_body,
        out_shape=(
            jax.ShapeDtypeStruct((NUM_LAYERS, EDGE_DIM, HID), jnp.float32),
            jax.ShapeDtypeStruct((NUM_LAYERS, HID), jnp.float32),
        ),
    )(edge_W, edge_b, lin_Ws, lin_bs)


def _encode_body(x_ref, w_ref, b_ref, bp0_ref, h_ref, ht_ref):
    h = (jnp.dot(x_ref[...], w_ref[...], preferred_element_type=jnp.float32)
         + b_ref[...])
    h_ref[...] = h
    ht_ref[...] = h + bp0_ref[...]


def _encode(xp, node_Wp, node_b, bp0):
    return pl.pallas_call(
        _encode_body,
        grid=(NRB,),
        in_specs=[
            pl.BlockSpec((RB, 16), lambda i: (i, 0)),
            pl.BlockSpec((16, HID), lambda i: (0, 0)),
            pl.BlockSpec((1, HID), lambda i: (0, 0)),
            pl.BlockSpec((1, HID), lambda i: (0, 0)),
        ],
        out_specs=(
            pl.BlockSpec((RB, HID), lambda i: (i, 0)),
            pl.BlockSpec((RB, HID), lambda i: (i, 0)),
        ),
        out_shape=(
            jax.ShapeDtypeStruct((N, HID), jnp.float32),
            jax.ShapeDtypeStruct((N, HID), jnp.float32),
        ),
    )(xp, node_Wp, node_b, bp0)


def _mlp_core(h, agg0, agg1, w1, b1, w2, b2, g, bln):
    z = h + agg0 + agg1
    u = jnp.maximum(
        jnp.dot(z, w1, preferred_element_type=jnp.float32) + b1, 0.0)
    v = jnp.dot(u, w2, preferred_element_type=jnp.float32) + b2
    m = jnp.mean(v, axis=1, keepdims=True)
    d = v - m
    var = jnp.mean(d * d, axis=1, keepdims=True)
    ln = d * lax.rsqrt(var + LN_EPS) * g + bln
    return jnp.maximum(ln, 0.0) + h


def _mlp_body(h_ref, a0_ref, a1_ref, w1_ref, b1_ref, w2_ref, b2_ref,
              g_ref, bln_ref, bpn_ref, h_out_ref, ht_out_ref):
    hn = _mlp_core(h_ref[...], a0_ref[...], a1_ref[...], w1_ref[...],
                   b1_ref[...], w2_ref[...], b2_ref[...], g_ref[...],
                   bln_ref[...])
    h_out_ref[...] = hn
    ht_out_ref[...] = hn + bpn_ref[...]


def _mlp(h, agg0, agg1, w1, b1, w2, b2, g, bln, bpn):
    full = lambda shape: pl.BlockSpec(shape, lambda i: tuple(0 for _ in shape))
    row = pl.BlockSpec((RB, HID), lambda i: (i, 0))
    return pl.pallas_call(
        _mlp_body,
        grid=(NRB,),
        in_specs=[
            row, row, row,
            full((HID, 2 * HID)), full((1, 2 * HID)),
            full((2 * HID, HID)), full((1, HID)),
            full((1, HID)), full((1, HID)), full((1, HID)),
        ],
        out_specs=(row, row),
        out_shape=(
            jax.ShapeDtypeStruct((N, HID), jnp.float32),
            jax.ShapeDtypeStruct((N, HID), jnp.float32),
        ),
    )(h, agg0, agg1, w1, b1, w2, b2, g, bln, bpn)


def _mlp_final_body(h_ref, a0_ref, a1_ref, w1_ref, b1_ref, w2_ref, b2_ref,
                    g_ref, bln_ref, batch_ref, out_ref):
    hn = _mlp_core(h_ref[...], a0_ref[...], a1_ref[...], w1_ref[...],
                   b1_ref[...], w2_ref[...], b2_ref[...], g_ref[...],
                   bln_ref[...])

    @pl.when(pl.program_id(0) == 0)
    def _():
        out_ref[...] = jnp.zeros_like(out_ref)

    bb = batch_ref[0]
    oh = (lax.broadcasted_iota(jnp.int32, (NUM_GRAPHS, RB), 0)
          == bb).astype(jnp.float32)
    out_ref[...] += jnp.dot(oh, hn, preferred_element_type=jnp.float32)


def _mlp_final(h, agg0, agg1, w1, b1, w2, b2, g, bln, batch3):
    full = lambda shape: pl.BlockSpec(shape, lambda i: tuple(0 for _ in shape))
    row = pl.BlockSpec((RB, HID), lambda i: (i, 0))
    return pl.pallas_call(
        _mlp_final_body,
        grid=(NRB,),
        in_specs=[
            row, row, row,
            full((HID, 2 * HID)), full((1, 2 * HID)),
            full((2 * HID, HID)), full((1, HID)),
            full((1, HID)), full((1, HID)),
            pl.BlockSpec((1, 1, RB), lambda i: (i, 0, 0)),
        ],
        out_specs=pl.BlockSpec((NUM_GRAPHS, HID), lambda i: (0, 0)),
        out_shape=jax.ShapeDtypeStruct((NUM_GRAPHS, HID), jnp.float32),
    )(h, agg0, agg1, w1, b1, w2, b2, g, bln, batch3)


def kernel(x, edge_index, edge_attr, batch, params):
    p = params
    lin_Ws = jnp.stack([lp['lin_W'] for lp in p['layers']])
    lin_bs = jnp.stack([lp['lin_b'] for lp in p['layers']])
    wp, bp = _prep(p['edge_W'], p['edge_b'].reshape(1, HID), lin_Ws, lin_bs)

    xp = jnp.pad(x, ((0, 0), (0, 16 - 9)))
    node_Wp = jnp.pad(p['node_W'], ((0, 16 - 9), (0, 0)))
    h, ht = _encode(xp, node_Wp, p['node_b'].reshape(1, HID), bp[0:1])

    src = edge_index[0]
    dst = edge_index[1]
    batch3 = batch.reshape(NRB, 1, RB)

    for l in range(NUM_LAYERS):
        lp = p['layers'][l]
        aggs = _sc_layer(ht, src, dst, edge_attr, wp[l])
        args = (h, aggs[0], aggs[1],
                lp['W1'], lp['b1'].reshape(1, 2 * HID),
                lp['W2'], lp['b2'].reshape(1, HID),
                lp['ln_g'].reshape(1, HID), lp['ln_b'].reshape(1, HID))
        if l < NUM_LAYERS - 1:
            h, ht = _mlp(*args, bp[l + 1:l + 2])
        else:
            out = _mlp_final(*args, batch3)
    return out


# speedup vs baseline: 3.7131x; 1.0369x over previous
import functools

import jax
import jax.numpy as jnp
from jax import lax
from jax.experimental import pallas as pl
from jax.experimental.pallas import tpu as pltpu
from jax.experimental.pallas import tpu_sc as plsc

N = 10000
E = 640000
HID = 128
NUM_LAYERS = 4
EDGE_DIM = 4
NUM_GRAPHS = 64
LN_EPS = 1e-5

NC = 2
NS = 16
LANES = 16
NW = NC * NS

EB = 128
BLK_W = 157
NBLK = NW * BLK_W
E_PAD = NBLK * EB
N_PAD = 10112
ROWS_PER_SUB = N_PAD // NS
DUMMY_DST = N_PAD - 1
NCHUNK = HID // LANES

RB = 1000
NRB = N // RB


def _sc_msg_body(ht_hbm, eidx_hbm, attr_hbm, wp_hbm, out_hbm,
                 agg_sh, eidx0_v, eidx1_v, attr0_v, attr1_v,
                 rows0_v, rows1_v, wp_v, gsem0, gsem1, ssem0, ssem1):
    gsem = (gsem0, gsem1)
    ssem = (ssem0, ssem1)
    core = lax.axis_index("c")
    sub = lax.axis_index("s")
    wid = sub * NC + core

    eidx_v = (eidx0_v, eidx1_v)
    attr_v = (attr0_v, attr1_v)
    rows_v = (rows0_v, rows1_v)

    zvec = jnp.zeros((LANES,), jnp.float32)

    def zero_row(r, carry):
        for c in range(NCHUNK):
            rows0_v[r, pl.ds(c * LANES, LANES)] = zvec
        return carry

    lax.fori_loop(0, EB, zero_row, 0)
    for j in range(ROWS_PER_SUB // EB):
        pltpu.sync_copy(
            rows0_v,
            agg_sh.at[pl.ds(sub * ROWS_PER_SUB + j * EB, EB)])
    rem = ROWS_PER_SUB - (ROWS_PER_SUB // EB) * EB
    if rem:
        pltpu.sync_copy(
            rows0_v.at[pl.ds(0, rem)],
            agg_sh.at[pl.ds(sub * ROWS_PER_SUB + (ROWS_PER_SUB // EB) * EB,
                            rem)])

    pltpu.sync_copy(wp_hbm, wp_v)
    wvec = [[wp_v[k, pl.ds(c * LANES, LANES)] for c in range(NCHUNK)]
            for k in range(EDGE_DIM)]

    plsc.subcore_barrier()

    def copy_idx(j, b):
        off = (wid + j * NW) * EB
        pltpu.sync_copy(eidx_hbm.at[:, pl.ds(off, EB)], eidx_v[b])
        pltpu.sync_copy(attr_hbm.at[pl.ds(off * EDGE_DIM, EB * EDGE_DIM)],
                        attr_v[b])

    def issue_gather(b):
        pltpu.async_copy(ht_hbm.at[eidx_v[b].at[0]], rows_v[b], gsem[b])

    def wait_gather(b):
        pltpu.make_async_copy(
            ht_hbm.at[eidx_v[b].at[0]], rows_v[b], gsem[b]).wait()

    def issue_scatter(b):
        pltpu.async_copy(rows_v[b], agg_sh.at[eidx_v[b].at[1]], ssem[b],
                         add=True)

    def wait_scatter(b):
        pltpu.make_async_copy(
            rows_v[b], agg_sh.at[eidx_v[b].at[1]], ssem[b]).wait()

    def compute(b):
        rv = rows_v[b]
        av = attr_v[b]

        def do_edge(e, ecarry):
            a = [plsc.load_gather(
                    av, [jnp.full((LANES,), e * EDGE_DIM + k, jnp.int32)])
                 for k in range(EDGE_DIM)]
            for c in range(NCHUNK):
                acc = rv[e, pl.ds(c * LANES, LANES)]
                for k in range(EDGE_DIM):
                    acc = acc + a[k] * wvec[k][c]
                rv[e, pl.ds(c * LANES, LANES)] = jnp.maximum(acc, 0.0)
            return ecarry

        lax.fori_loop(0, EB, do_edge, 0, unroll=2)

    def step(j, b, first, last):
        if not first:
            wait_scatter(1 - b)
        if not last:
            copy_idx(j + 1, 1 - b)
            issue_gather(1 - b)
        wait_gather(b)
        compute(b)
        issue_scatter(b)

    copy_idx(0, 0)
    issue_gather(0)
    step(0, 0, first=True, last=False)

    def pair(p, carry):
        j = 1 + 2 * p
        step(j, 1, first=False, last=False)
        step(j + 1, 0, first=False, last=False)
        return carry

    lax.fori_loop(0, (BLK_W - 3) // 2, pair, 0)
    step(BLK_W - 2, 1, first=False, last=False)
    step(BLK_W - 1, 0, first=False, last=True)
    wait_scatter(0)

    plsc.subcore_barrier()

    pltpu.sync_copy(
        agg_sh.at[pl.ds(sub * ROWS_PER_SUB, ROWS_PER_SUB)],
        out_hbm.at[core, pl.ds(sub * ROWS_PER_SUB, ROWS_PER_SUB)])


_sc_layer = pl.kernel(
    _sc_msg_body,
    out_type=jax.ShapeDtypeStruct((NC, N_PAD, HID), jnp.float32),
    mesh=plsc.VectorSubcoreMesh(core_axis_name="c", subcore_axis_name="s"),
    scratch_types=[
        pltpu.VMEM_SHARED((N_PAD, HID), jnp.float32),
        pltpu.VMEM((2, EB), jnp.int32),
        pltpu.VMEM((2, EB), jnp.int32),
        pltpu.VMEM((EB * EDGE_DIM,), jnp.float32),
        pltpu.VMEM((EB * EDGE_DIM,), jnp.float32),
        pltpu.VMEM((EB, HID), jnp.float32),
        pltpu.VMEM((EB, HID), jnp.float32),
        pltpu.VMEM((EDGE_DIM, HID), jnp.float32),
        pltpu.SemaphoreType.DMA,
        pltpu.SemaphoreType.DMA,
        pltpu.SemaphoreType.DMA,
        pltpu.SemaphoreType.DMA,
    ],
    compiler_params=pltpu.CompilerParams(needs_layout_passes=False),
)


def _prep_body(edge_W_ref, edge_b_ref, lin_Ws_ref, lin_bs_ref,
               wp_ref, bp_ref):
    ew = edge_W_ref[...]
    eb = edge_b_ref[...]
    for l in range(NUM_LAYERS):
        lw = lin_Ws_ref[l]
        wp_ref[l] = jnp.dot(ew, lw, preferred_element_type=jnp.float32)
        bp_ref[pl.ds(l, 1), :] = (
            jnp.dot(eb, lw, preferred_element_type=jnp.float32)
            + lin_bs_ref[pl.ds(l, 1), :])


def _prep(edge_W, edge_b, lin_Ws, lin_bs):
    return pl.pallas_call(
        _prep_body,
        out_shape=(
            jax.ShapeDtypeStruct((NUM_LAYERS, EDGE_DIM, HID), jnp.float32),
            jax.ShapeDtypeStruct((NUM_LAYERS, HID), jnp.float32),
        ),
    )(edge_W, edge_b, lin_Ws, lin_bs)


def _encode_body(x_ref, w_ref, b_ref, bp0_ref, h_ref, ht_ref):
    h = (jnp.dot(x_ref[...], w_ref[...], preferred_element_type=jnp.float32)
         + b_ref[...])
    h_ref[...] = h
    ht_ref[...] = h + bp0_ref[...]


def _encode(xp, node_Wp, node_b, bp0):
    return pl.pallas_call(
        _encode_body,
        grid=(NRB,),
        in_specs=[
            pl.BlockSpec((RB, 16), lambda i: (i, 0)),
            pl.BlockSpec((16, HID), lambda i: (0, 0)),
            pl.BlockSpec((1, HID), lambda i: (0, 0)),
            pl.BlockSpec((1, HID), lambda i: (0, 0)),
        ],
        out_specs=(
            pl.BlockSpec((RB, HID), lambda i: (i, 0)),
            pl.BlockSpec((RB, HID), lambda i: (i, 0)),
        ),
        out_shape=(
            jax.ShapeDtypeStruct((N, HID), jnp.float32),
            jax.ShapeDtypeStruct((N, HID), jnp.float32),
        ),
    )(xp, node_Wp, node_b, bp0)


def _mlp_core(h, agg0, agg1, w1, b1, w2, b2, g, bln):
    z = h + agg0 + agg1
    u = jnp.maximum(
        jnp.dot(z, w1, preferred_element_type=jnp.float32) + b1, 0.0)
    v = jnp.dot(u, w2, preferred_element_type=jnp.float32) + b2
    m = jnp.mean(v, axis=1, keepdims=True)
    d = v - m
    var = jnp.mean(d * d, axis=1, keepdims=True)
    ln = d * lax.rsqrt(var + LN_EPS) * g + bln
    return jnp.maximum(ln, 0.0) + h


def _mlp_body(h_ref, a0_ref, a1_ref, w1_ref, b1_ref, w2_ref, b2_ref,
              g_ref, bln_ref, bpn_ref, h_out_ref, ht_out_ref):
    hn = _mlp_core(h_ref[...], a0_ref[...], a1_ref[...], w1_ref[...],
                   b1_ref[...], w2_ref[...], b2_ref[...], g_ref[...],
                   bln_ref[...])
    h_out_ref[...] = hn
    ht_out_ref[...] = hn + bpn_ref[...]


def _mlp(h, agg0, agg1, w1, b1, w2, b2, g, bln, bpn):
    full = lambda shape: pl.BlockSpec(shape, lambda i: tuple(0 for _ in shape))
    row = pl.BlockSpec((RB, HID), lambda i: (i, 0))
    return pl.pallas_call(
        _mlp_body,
        grid=(NRB,),
        in_specs=[
            row, row, row,
            full((HID, 2 * HID)), full((1, 2 * HID)),
            full((2 * HID, HID)), full((1, HID)),
            full((1, HID)), full((1, HID)), full((1, HID)),
        ],
        out_specs=(row, row),
        out_shape=(
            jax.ShapeDtypeStruct((N, HID), jnp.float32),
            jax.ShapeDtypeStruct((N, HID), jnp.float32),
        ),
    )(h, agg0, agg1, w1, b1, w2, b2, g, bln, bpn)


def _mlp_final_body(h_ref, a0_ref, a1_ref, w1_ref, b1_ref, w2_ref, b2_ref,
                    g_ref, bln_ref, batch_ref, out_ref):
    hn = _mlp_core(h_ref[...], a0_ref[...], a1_ref[...], w1_ref[...],
                   b1_ref[...], w2_ref[...], b2_ref[...], g_ref[...],
                   bln_ref[...])

    @pl.when(pl.program_id(0) == 0)
    def _():
        out_ref[...] = jnp.zeros_like(out_ref)

    bb = batch_ref[0]
    oh = (lax.broadcasted_iota(jnp.int32, (NUM_GRAPHS, RB), 0)
          == bb).astype(jnp.float32)
    out_ref[...] += jnp.dot(oh, hn, preferred_element_type=jnp.float32)


def _mlp_final(h, agg0, agg1, w1, b1, w2, b2, g, bln, batch3):
    full = lambda shape: pl.BlockSpec(shape, lambda i: tuple(0 for _ in shape))
    row = pl.BlockSpec((RB, HID), lambda i: (i, 0))
    return pl.pallas_call(
        _mlp_final_body,
        grid=(NRB,),
        in_specs=[
            row, row, row,
            full((HID, 2 * HID)), full((1, 2 * HID)),
            full((2 * HID, HID)), full((1, HID)),
            full((1, HID)), full((1, HID)),
            pl.BlockSpec((1, 1, RB), lambda i: (i, 0, 0)),
        ],
        out_specs=pl.BlockSpec((NUM_GRAPHS, HID), lambda i: (0, 0)),
        out_shape=jax.ShapeDtypeStruct((NUM_GRAPHS, HID), jnp.float32),
    )(h, agg0, agg1, w1, b1, w2, b2, g, bln, batch3)


def kernel(x, edge_index, edge_attr, batch, params):
    p = params
    lin_Ws = jnp.stack([lp['lin_W'] for lp in p['layers']])
    lin_bs = jnp.stack([lp['lin_b'] for lp in p['layers']])
    wp, bp = _prep(p['edge_W'], p['edge_b'].reshape(1, HID), lin_Ws, lin_bs)

    xp = jnp.pad(x, ((0, 0), (0, 16 - 9)))
    node_Wp = jnp.pad(p['node_W'], ((0, 16 - 9), (0, 0)))
    h, ht = _encode(xp, node_Wp, p['node_b'].reshape(1, HID), bp[0:1])

    pad_e = E_PAD - E
    pad_cols = jnp.stack([jnp.zeros((pad_e,), jnp.int32),
                          jnp.full((pad_e,), DUMMY_DST, jnp.int32)])
    eidx = jnp.concatenate([edge_index, pad_cols], axis=1)
    attr_p = jnp.pad(edge_attr, ((0, pad_e), (0, 0))).reshape(-1)
    batch3 = batch.reshape(NRB, 1, RB)

    for l in range(NUM_LAYERS):
        lp = p['layers'][l]
        aggs = _sc_layer(ht, eidx, attr_p, wp[l])
        args = (h, aggs[0], aggs[1],
                lp['W1'], lp['b1'].reshape(1, 2 * HID),
                lp['W2'], lp['b2'].reshape(1, HID),
                lp['ln_g'].reshape(1, HID), lp['ln_b'].reshape(1, HID))
        if l < NUM_LAYERS - 1:
            h, ht = _mlp(*args, bp[l + 1:l + 2])
        else:
            out = _mlp_final(*args, batch3)
    return out


# speedup vs baseline: 4.0754x; 1.0976x over previous
import functools

import jax
import jax.numpy as jnp
from jax import lax
from jax.experimental import pallas as pl
from jax.experimental.pallas import tpu as pltpu
from jax.experimental.pallas import tpu_sc as plsc

N = 10000
E = 640000
HID = 128
NUM_LAYERS = 4
EDGE_DIM = 4
NUM_GRAPHS = 64
LN_EPS = 1e-5

NC = 2
NS = 16
LANES = 16
NW = NC * NS

EB = 128
BLK_W = 157
NBLK = NW * BLK_W
E_PAD = NBLK * EB
N_PAD = 10112
ROWS_PER_SUB = N_PAD // NS
DUMMY_DST = N_PAD - 1
NCHUNK = HID // LANES

RB = 1000
NRB = N // RB


def _sc_msg_body(ht_hbm, eidx_hbm, attr_hbm, wp_hbm, out_hbm,
                 agg_sh, eidx0_v, eidx1_v, attr0_v, attr1_v,
                 rows0_v, rows1_v, wp_v, gsem0, gsem1, ssem0, ssem1):
    gsem = (gsem0, gsem1)
    ssem = (ssem0, ssem1)
    core = lax.axis_index("c")
    sub = lax.axis_index("s")
    wid = sub * NC + core

    eidx_v = (eidx0_v, eidx1_v)
    attr_v = (attr0_v, attr1_v)
    rows_v = (rows0_v, rows1_v)

    zvec = jnp.zeros((LANES,), jnp.float32)

    def zero_row(r, carry):
        for c in range(NCHUNK):
            rows0_v[r, pl.ds(c * LANES, LANES)] = zvec
        return carry

    lax.fori_loop(0, EB, zero_row, 0)
    for j in range(ROWS_PER_SUB // EB):
        pltpu.sync_copy(
            rows0_v,
            agg_sh.at[pl.ds(sub * ROWS_PER_SUB + j * EB, EB)])
    rem = ROWS_PER_SUB - (ROWS_PER_SUB // EB) * EB
    if rem:
        pltpu.sync_copy(
            rows0_v.at[pl.ds(0, rem)],
            agg_sh.at[pl.ds(sub * ROWS_PER_SUB + (ROWS_PER_SUB // EB) * EB,
                            rem)])

    pltpu.sync_copy(wp_hbm, wp_v)
    wvec = [[wp_v[k, pl.ds(c * LANES, LANES)] for c in range(NCHUNK)]
            for k in range(EDGE_DIM)]

    plsc.subcore_barrier()

    def copy_idx(j, b):
        off = (wid + j * NW) * EB
        pltpu.sync_copy(eidx_hbm.at[:, pl.ds(off, EB)], eidx_v[b])
        pltpu.sync_copy(attr_hbm.at[pl.ds(off * EDGE_DIM, EB * EDGE_DIM)],
                        attr_v[b])

    def issue_gather(b):
        pltpu.async_copy(ht_hbm.at[eidx_v[b].at[0]], rows_v[b], gsem[b])

    def wait_gather(b):
        pltpu.make_async_copy(
            ht_hbm.at[eidx_v[b].at[0]], rows_v[b], gsem[b]).wait()

    def issue_scatter(b):
        pltpu.async_copy(rows_v[b], agg_sh.at[eidx_v[b].at[1]], ssem[b],
                         add=True)

    def wait_scatter(b):
        pltpu.make_async_copy(
            rows_v[b], agg_sh.at[eidx_v[b].at[1]], ssem[b]).wait()

    def compute(b):
        rv = rows_v[b]
        av = attr_v[b]

        def lane_bcast(vec, lane):
            return lax.gather(
                vec, jnp.full((LANES, 1), lane, jnp.int32),
                lax.GatherDimensionNumbers((), (0,), (0,)), (1,),
                mode=lax.GatherScatterMode.PROMISE_IN_BOUNDS)

        def do_grp(q, ecarry):
            av16 = av[pl.ds(q * (4 * EDGE_DIM), LANES)]
            for t in range(4):
                e = q * 4 + t
                a = [lane_bcast(av16, t * EDGE_DIM + k)
                     for k in range(EDGE_DIM)]
                for c in range(NCHUNK):
                    acc = rv[e, pl.ds(c * LANES, LANES)]
                    for k in range(EDGE_DIM):
                        acc = acc + a[k] * wvec[k][c]
                    rv[e, pl.ds(c * LANES, LANES)] = jnp.maximum(acc, 0.0)
            return ecarry

        lax.fori_loop(0, EB // 4, do_grp, 0)

    def step(j, b, first, last):
        if not first:
            wait_scatter(1 - b)
        if not last:
            copy_idx(j + 1, 1 - b)
            issue_gather(1 - b)
        wait_gather(b)
        compute(b)
        issue_scatter(b)

    copy_idx(0, 0)
    issue_gather(0)
    step(0, 0, first=True, last=False)

    def pair(p, carry):
        j = 1 + 2 * p
        step(j, 1, first=False, last=False)
        step(j + 1, 0, first=False, last=False)
        return carry

    lax.fori_loop(0, (BLK_W - 3) // 2, pair, 0)
    step(BLK_W - 2, 1, first=False, last=False)
    step(BLK_W - 1, 0, first=False, last=True)
    wait_scatter(0)

    plsc.subcore_barrier()

    pltpu.sync_copy(
        agg_sh.at[pl.ds(sub * ROWS_PER_SUB, ROWS_PER_SUB)],
        out_hbm.at[core, pl.ds(sub * ROWS_PER_SUB, ROWS_PER_SUB)])


_sc_layer = pl.kernel(
    _sc_msg_body,
    out_type=jax.ShapeDtypeStruct((NC, N_PAD, HID), jnp.float32),
    mesh=plsc.VectorSubcoreMesh(core_axis_name="c", subcore_axis_name="s"),
    scratch_types=[
        pltpu.VMEM_SHARED((N_PAD, HID), jnp.float32),
        pltpu.VMEM((2, EB), jnp.int32),
        pltpu.VMEM((2, EB), jnp.int32),
        pltpu.VMEM((EB * EDGE_DIM,), jnp.float32),
        pltpu.VMEM((EB * EDGE_DIM,), jnp.float32),
        pltpu.VMEM((EB, HID), jnp.float32),
        pltpu.VMEM((EB, HID), jnp.float32),
        pltpu.VMEM((EDGE_DIM, HID), jnp.float32),
        pltpu.SemaphoreType.DMA,
        pltpu.SemaphoreType.DMA,
        pltpu.SemaphoreType.DMA,
        pltpu.SemaphoreType.DMA,
    ],
    compiler_params=pltpu.CompilerParams(needs_layout_passes=False),
)


def _prep_body(edge_W_ref, edge_b_ref, lin_Ws_ref, lin_bs_ref,
               wp_ref, bp_ref):
    ew = edge_W_ref[...]
    eb = edge_b_ref[...]
    for l in range(NUM_LAYERS):
        lw = lin_Ws_ref[l]
        wp_ref[l] = jnp.dot(ew, lw, preferred_element_type=jnp.float32)
        bp_ref[pl.ds(l, 1), :] = (
            jnp.dot(eb, lw, preferred_element_type=jnp.float32)
            + lin_bs_ref[pl.ds(l, 1), :])


def _prep(edge_W, edge_b, lin_Ws, lin_bs):
    return pl.pallas_call(
        _prep_body,
        out_shape=(
            jax.ShapeDtypeStruct((NUM_LAYERS, EDGE_DIM, HID), jnp.float32),
            jax.ShapeDtypeStruct((NUM_LAYERS, HID), jnp.float32),
        ),
    )(edge_W, edge_b, lin_Ws, lin_bs)


def _encode_body(x_ref, w_ref, b_ref, bp0_ref, h_ref, ht_ref):
    h = (jnp.dot(x_ref[...], w_ref[...], preferred_element_type=jnp.float32)
         + b_ref[...])
    h_ref[...] = h
    ht_ref[...] = h + bp0_ref[...]


def _encode(xp, node_Wp, node_b, bp0):
    return pl.pallas_call(
        _encode_body,
        grid=(NRB,),
        in_specs=[
            pl.BlockSpec((RB, 16), lambda i: (i, 0)),
            pl.BlockSpec((16, HID), lambda i: (0, 0)),
            pl.BlockSpec((1, HID), lambda i: (0, 0)),
            pl.BlockSpec((1, HID), lambda i: (0, 0)),
        ],
        out_specs=(
            pl.BlockSpec((RB, HID), lambda i: (i, 0)),
            pl.BlockSpec((RB, HID), lambda i: (i, 0)),
        ),
        out_shape=(
            jax.ShapeDtypeStruct((N, HID), jnp.float32),
            jax.ShapeDtypeStruct((N, HID), jnp.float32),
        ),
    )(xp, node_Wp, node_b, bp0)


def _mlp_core(h, agg0, agg1, w1, b1, w2, b2, g, bln):
    z = h + agg0 + agg1
    u = jnp.maximum(
        jnp.dot(z, w1, preferred_element_type=jnp.float32) + b1, 0.0)
    v = jnp.dot(u, w2, preferred_element_type=jnp.float32) + b2
    m = jnp.mean(v, axis=1, keepdims=True)
    d = v - m
    var = jnp.mean(d * d, axis=1, keepdims=True)
    ln = d * lax.rsqrt(var + LN_EPS) * g + bln
    return jnp.maximum(ln, 0.0) + h


def _mlp_body(h_ref, a0_ref, a1_ref, w1_ref, b1_ref, w2_ref, b2_ref,
              g_ref, bln_ref, bpn_ref, h_out_ref, ht_out_ref):
    hn = _mlp_core(h_ref[...], a0_ref[...], a1_ref[...], w1_ref[...],
                   b1_ref[...], w2_ref[...], b2_ref[...], g_ref[...],
                   bln_ref[...])
    h_out_ref[...] = hn
    ht_out_ref[...] = hn + bpn_ref[...]


def _mlp(h, agg0, agg1, w1, b1, w2, b2, g, bln, bpn):
    full = lambda shape: pl.BlockSpec(shape, lambda i: tuple(0 for _ in shape))
    row = pl.BlockSpec((RB, HID), lambda i: (i, 0))
    return pl.pallas_call(
        _mlp_body,
        grid=(NRB,),
        in_specs=[
            row, row, row,
            full((HID, 2 * HID)), full((1, 2 * HID)),
            full((2 * HID, HID)), full((1, HID)),
            full((1, HID)), full((1, HID)), full((1, HID)),
        ],
        out_specs=(row, row),
        out_shape=(
            jax.ShapeDtypeStruct((N, HID), jnp.float32),
            jax.ShapeDtypeStruct((N, HID), jnp.float32),
        ),
    )(h, agg0, agg1, w1, b1, w2, b2, g, bln, bpn)


def _mlp_final_body(h_ref, a0_ref, a1_ref, w1_ref, b1_ref, w2_ref, b2_ref,
                    g_ref, bln_ref, batch_ref, out_ref):
    hn = _mlp_core(h_ref[...], a0_ref[...], a1_ref[...], w1_ref[...],
                   b1_ref[...], w2_ref[...], b2_ref[...], g_ref[...],
                   bln_ref[...])

    @pl.when(pl.program_id(0) == 0)
    def _():
        out_ref[...] = jnp.zeros_like(out_ref)

    bb = batch_ref[0]
    oh = (lax.broadcasted_iota(jnp.int32, (NUM_GRAPHS, RB), 0)
          == bb).astype(jnp.float32)
    out_ref[...] += jnp.dot(oh, hn, preferred_element_type=jnp.float32)


def _mlp_final(h, agg0, agg1, w1, b1, w2, b2, g, bln, batch3):
    full = lambda shape: pl.BlockSpec(shape, lambda i: tuple(0 for _ in shape))
    row = pl.BlockSpec((RB, HID), lambda i: (i, 0))
    return pl.pallas_call(
        _mlp_final_body,
        grid=(NRB,),
        in_specs=[
            row, row, row,
            full((HID, 2 * HID)), full((1, 2 * HID)),
            full((2 * HID, HID)), full((1, HID)),
            full((1, HID)), full((1, HID)),
            pl.BlockSpec((1, 1, RB), lambda i: (i, 0, 0)),
        ],
        out_specs=pl.BlockSpec((NUM_GRAPHS, HID), lambda i: (0, 0)),
        out_shape=jax.ShapeDtypeStruct((NUM_GRAPHS, HID), jnp.float32),
    )(h, agg0, agg1, w1, b1, w2, b2, g, bln, batch3)


def kernel(x, edge_index, edge_attr, batch, params):
    p = params
    lin_Ws = jnp.stack([lp['lin_W'] for lp in p['layers']])
    lin_bs = jnp.stack([lp['lin_b'] for lp in p['layers']])
    wp, bp = _prep(p['edge_W'], p['edge_b'].reshape(1, HID), lin_Ws, lin_bs)

    xp = jnp.pad(x, ((0, 0), (0, 16 - 9)))
    node_Wp = jnp.pad(p['node_W'], ((0, 16 - 9), (0, 0)))
    h, ht = _encode(xp, node_Wp, p['node_b'].reshape(1, HID), bp[0:1])

    pad_e = E_PAD - E
    pad_cols = jnp.stack([jnp.zeros((pad_e,), jnp.int32),
                          jnp.full((pad_e,), DUMMY_DST, jnp.int32)])
    eidx = jnp.concatenate([edge_index, pad_cols], axis=1)
    attr_p = jnp.pad(edge_attr, ((0, pad_e), (0, 0))).reshape(-1)
    batch3 = batch.reshape(NRB, 1, RB)

    for l in range(NUM_LAYERS):
        lp = p['layers'][l]
        aggs = _sc_layer(ht, eidx, attr_p, wp[l])
        args = (h, aggs[0], aggs[1],
                lp['W1'], lp['b1'].reshape(1, 2 * HID),
                lp['W2'], lp['b2'].reshape(1, HID),
                lp['ln_g'].reshape(1, HID), lp['ln_b'].reshape(1, HID))
        if l < NUM_LAYERS - 1:
            h, ht = _mlp(*args, bp[l + 1:l + 2])
        else:
            out = _mlp_final(*args, batch3)
    return out


# speedup vs baseline: 4.4560x; 1.0934x over previous
import functools

import jax
import jax.numpy as jnp
from jax import lax
from jax.experimental import pallas as pl
from jax.experimental.pallas import tpu as pltpu
from jax.experimental.pallas import tpu_sc as plsc

N = 10000
E = 640000
HID = 128
NUM_LAYERS = 4
EDGE_DIM = 4
NUM_GRAPHS = 64
LN_EPS = 1e-5

NC = 2
NS = 16
LANES = 16
NW = NC * NS

EB = 128
BLK_W = 157
NBLK = NW * BLK_W
E_PAD = NBLK * EB
N_PAD = 10112
ROWS_PER_SUB = N_PAD // NS
DUMMY_DST = N_PAD - 1
NCHUNK = HID // LANES

RB = 1000
NRB = N // RB


def _sc_msg_body(ht_hbm, eidx_hbm, attr_hbm, wp_hbm, out_hbm,
                 agg_sh, eidx0_v, eidx1_v, attr0_v, attr1_v,
                 rows0_v, rows1_v, wp_v,
                 gsem0, gsem1, ssem0, ssem1,
                 isem0, isem1, asem0, asem1):
    gsem = (gsem0, gsem1)
    ssem = (ssem0, ssem1)
    isem = (isem0, isem1)
    asem = (asem0, asem1)
    core = lax.axis_index("c")
    sub = lax.axis_index("s")
    wid = sub * NC + core

    eidx_v = (eidx0_v, eidx1_v)
    attr_v = (attr0_v, attr1_v)
    rows_v = (rows0_v, rows1_v)

    zvec = jnp.zeros((LANES,), jnp.float32)

    def zero_row(r, carry):
        for c in range(NCHUNK):
            rows0_v[r, pl.ds(c * LANES, LANES)] = zvec
        return carry

    lax.fori_loop(0, EB, zero_row, 0)
    for j in range(ROWS_PER_SUB // EB):
        pltpu.sync_copy(
            rows0_v,
            agg_sh.at[pl.ds(sub * ROWS_PER_SUB + j * EB, EB)])
    rem = ROWS_PER_SUB - (ROWS_PER_SUB // EB) * EB
    if rem:
        pltpu.sync_copy(
            rows0_v.at[pl.ds(0, rem)],
            agg_sh.at[pl.ds(sub * ROWS_PER_SUB + (ROWS_PER_SUB // EB) * EB,
                            rem)])

    pltpu.sync_copy(wp_hbm, wp_v)
    wvec = [[wp_v[k, pl.ds(c * LANES, LANES)] for c in range(NCHUNK)]
            for k in range(EDGE_DIM)]

    plsc.subcore_barrier()

    def issue_idx(j, b):
        blk = wid * BLK_W + j
        pltpu.async_copy(eidx_hbm.at[pl.ds(blk * 2, 2)], eidx_v[b], isem[b])
        pltpu.async_copy(
            attr_hbm.at[pl.ds(blk * EB * EDGE_DIM, EB * EDGE_DIM)],
            attr_v[b], asem[b])

    def wait_idx(j, b):
        blk = wid * BLK_W + j
        pltpu.make_async_copy(eidx_hbm.at[pl.ds(blk * 2, 2)], eidx_v[b],
                              isem[b]).wait()
        pltpu.make_async_copy(
            attr_hbm.at[pl.ds(blk * EB * EDGE_DIM, EB * EDGE_DIM)],
            attr_v[b], asem[b]).wait()

    def issue_gather(b):
        pltpu.async_copy(ht_hbm.at[eidx_v[b].at[0]], rows_v[b], gsem[b])

    def wait_gather(b):
        pltpu.make_async_copy(
            ht_hbm.at[eidx_v[b].at[0]], rows_v[b], gsem[b]).wait()

    def issue_scatter(b):
        pltpu.async_copy(rows_v[b], agg_sh.at[eidx_v[b].at[1]], ssem[b],
                         add=True)

    def wait_scatter(b):
        pltpu.make_async_copy(
            rows_v[b], agg_sh.at[eidx_v[b].at[1]], ssem[b]).wait()

    def lane_bcast(vec, lane):
        return lax.gather(
            vec, jnp.full((LANES, 1), lane, jnp.int32),
            lax.GatherDimensionNumbers((), (0,), (0,)), (1,),
            mode=lax.GatherScatterMode.PROMISE_IN_BOUNDS)

    def compute(b):
        rv = rows_v[b]
        av = attr_v[b]

        def do_grp(q, ecarry):
            av16 = av[pl.ds(q * (4 * EDGE_DIM), LANES)]
            for t in range(4):
                e = q * 4 + t
                a = [lane_bcast(av16, t * EDGE_DIM + k)
                     for k in range(EDGE_DIM)]
                for c in range(NCHUNK):
                    acc = rv[e, pl.ds(c * LANES, LANES)]
                    for k in range(EDGE_DIM):
                        acc = acc + a[k] * wvec[k][c]
                    rv[e, pl.ds(c * LANES, LANES)] = jnp.maximum(acc, 0.0)
            return ecarry

        lax.fori_loop(0, EB // 4, do_grp, 0)

    def step(j, jm, first=False, has1=True):
        br = jm % 2
        if not first:
            wait_scatter(1 - br)
        if has1:
            issue_idx(j + 1, 1 - br)
            wait_idx(j + 1, 1 - br)
            issue_gather(1 - br)
        wait_gather(br)
        compute(br)
        issue_scatter(br)

    issue_idx(0, 0)
    wait_idx(0, 0)
    issue_gather(0)
    step(0, 0, first=True)

    def pair(p, carry):
        j = 1 + 2 * p
        step(j, 1)
        step(j + 1, 0)
        return carry

    lax.fori_loop(0, (BLK_W - 3) // 2, pair, 0)
    step(BLK_W - 2, 1)
    step(BLK_W - 1, 0, has1=False)
    wait_scatter((BLK_W - 1) % 2)

    plsc.subcore_barrier()

    pltpu.sync_copy(
        agg_sh.at[pl.ds(sub * ROWS_PER_SUB, ROWS_PER_SUB)],
        out_hbm.at[core, pl.ds(sub * ROWS_PER_SUB, ROWS_PER_SUB)])


_sc_layer = pl.kernel(
    _sc_msg_body,
    out_type=jax.ShapeDtypeStruct((NC, N_PAD, HID), jnp.float32),
    mesh=plsc.VectorSubcoreMesh(core_axis_name="c", subcore_axis_name="s"),
    scratch_types=[
        pltpu.VMEM_SHARED((N_PAD, HID), jnp.float32),
        pltpu.VMEM((2, EB), jnp.int32),
        pltpu.VMEM((2, EB), jnp.int32),
        pltpu.VMEM((EB * EDGE_DIM,), jnp.float32),
        pltpu.VMEM((EB * EDGE_DIM,), jnp.float32),
        pltpu.VMEM((EB, HID), jnp.float32),
        pltpu.VMEM((EB, HID), jnp.float32),
        pltpu.VMEM((EDGE_DIM, HID), jnp.float32),
        pltpu.SemaphoreType.DMA,
        pltpu.SemaphoreType.DMA,
        pltpu.SemaphoreType.DMA,
        pltpu.SemaphoreType.DMA,
        pltpu.SemaphoreType.DMA,
        pltpu.SemaphoreType.DMA,
        pltpu.SemaphoreType.DMA,
        pltpu.SemaphoreType.DMA,
    ],
    compiler_params=pltpu.CompilerParams(needs_layout_passes=False),
)


def _prep_body(edge_W_ref, edge_b_ref, lin_Ws_ref, lin_bs_ref,
               wp_ref, bp_ref):
    ew = edge_W_ref[...]
    eb = edge_b_ref[...]
    for l in range(NUM_LAYERS):
        lw = lin_Ws_ref[l]
        wp_ref[l] = jnp.dot(ew, lw, preferred_element_type=jnp.float32)
        bp_ref[pl.ds(l, 1), :] = (
            jnp.dot(eb, lw, preferred_element_type=jnp.float32)
            + lin_bs_ref[pl.ds(l, 1), :])


def _prep(edge_W, edge_b, lin_Ws, lin_bs):
    return pl.pallas_call(
        _prep_body,
        out_shape=(
            jax.ShapeDtypeStruct((NUM_LAYERS, EDGE_DIM, HID), jnp.float32),
            jax.ShapeDtypeStruct((NUM_LAYERS, HID), jnp.float32),
        ),
    )(edge_W, edge_b, lin_Ws, lin_bs)


def _encode_body(x_ref, w_ref, b_ref, bp0_ref, h_ref, ht_ref):
    h = (jnp.dot(x_ref[...], w_ref[...], preferred_element_type=jnp.float32)
         + b_ref[...])
    h_ref[...] = h
    ht_ref[...] = h + bp0_ref[...]


def _encode(xp, node_Wp, node_b, bp0):
    return pl.pallas_call(
        _encode_body,
        grid=(NRB,),
        in_specs=[
            pl.BlockSpec((RB, 16), lambda i: (i, 0)),
            pl.BlockSpec((16, HID), lambda i: (0, 0)),
            pl.BlockSpec((1, HID), lambda i: (0, 0)),
            pl.BlockSpec((1, HID), lambda i: (0, 0)),
        ],
        out_specs=(
            pl.BlockSpec((RB, HID), lambda i: (i, 0)),
            pl.BlockSpec((RB, HID), lambda i: (i, 0)),
        ),
        out_shape=(
            jax.ShapeDtypeStruct((N, HID), jnp.float32),
            jax.ShapeDtypeStruct((N, HID), jnp.float32),
        ),
    )(xp, node_Wp, node_b, bp0)


def _mlp_core(h, agg0, agg1, w1, b1, w2, b2, g, bln):
    z = h + agg0 + agg1
    u = jnp.maximum(
        jnp.dot(z, w1, preferred_element_type=jnp.float32) + b1, 0.0)
    v = jnp.dot(u, w2, preferred_element_type=jnp.float32) + b2
    m = jnp.mean(v, axis=1, keepdims=True)
    d = v - m
    var = jnp.mean(d * d, axis=1, keepdims=True)
    ln = d * lax.rsqrt(var + LN_EPS) * g + bln
    return jnp.maximum(ln, 0.0) + h


def _mlp_body(h_ref, a0_ref, a1_ref, w1_ref, b1_ref, w2_ref, b2_ref,
              g_ref, bln_ref, bpn_ref, h_out_ref, ht_out_ref):
    hn = _mlp_core(h_ref[...], a0_ref[...], a1_ref[...], w1_ref[...],
                   b1_ref[...], w2_ref[...], b2_ref[...], g_ref[...],
                   bln_ref[...])
    h_out_ref[...] = hn
    ht_out_ref[...] = hn + bpn_ref[...]


def _mlp(h, agg0, agg1, w1, b1, w2, b2, g, bln, bpn):
    full = lambda shape: pl.BlockSpec(shape, lambda i: tuple(0 for _ in shape))
    row = pl.BlockSpec((RB, HID), lambda i: (i, 0))
    return pl.pallas_call(
        _mlp_body,
        grid=(NRB,),
        in_specs=[
            row, row, row,
            full((HID, 2 * HID)), full((1, 2 * HID)),
            full((2 * HID, HID)), full((1, HID)),
            full((1, HID)), full((1, HID)), full((1, HID)),
        ],
        out_specs=(row, row),
        out_shape=(
            jax.ShapeDtypeStruct((N, HID), jnp.float32),
            jax.ShapeDtypeStruct((N, HID), jnp.float32),
        ),
    )(h, agg0, agg1, w1, b1, w2, b2, g, bln, bpn)


def _mlp_final_body(h_ref, a0_ref, a1_ref, w1_ref, b1_ref, w2_ref, b2_ref,
                    g_ref, bln_ref, batch_ref, out_ref):
    hn = _mlp_core(h_ref[...], a0_ref[...], a1_ref[...], w1_ref[...],
                   b1_ref[...], w2_ref[...], b2_ref[...], g_ref[...],
                   bln_ref[...])

    @pl.when(pl.program_id(0) == 0)
    def _():
        out_ref[...] = jnp.zeros_like(out_ref)

    bb = batch_ref[0]
    oh = (lax.broadcasted_iota(jnp.int32, (NUM_GRAPHS, RB), 0)
          == bb).astype(jnp.float32)
    out_ref[...] += jnp.dot(oh, hn, preferred_element_type=jnp.float32)


def _mlp_final(h, agg0, agg1, w1, b1, w2, b2, g, bln, batch3):
    full = lambda shape: pl.BlockSpec(shape, lambda i: tuple(0 for _ in shape))
    row = pl.BlockSpec((RB, HID), lambda i: (i, 0))
    return pl.pallas_call(
        _mlp_final_body,
        grid=(NRB,),
        in_specs=[
            row, row, row,
            full((HID, 2 * HID)), full((1, 2 * HID)),
            full((2 * HID, HID)), full((1, HID)),
            full((1, HID)), full((1, HID)),
            pl.BlockSpec((1, 1, RB), lambda i: (i, 0, 0)),
        ],
        out_specs=pl.BlockSpec((NUM_GRAPHS, HID), lambda i: (0, 0)),
        out_shape=jax.ShapeDtypeStruct((NUM_GRAPHS, HID), jnp.float32),
    )(h, agg0, agg1, w1, b1, w2, b2, g, bln, batch3)


def kernel(x, edge_index, edge_attr, batch, params):
    p = params
    lin_Ws = jnp.stack([lp['lin_W'] for lp in p['layers']])
    lin_bs = jnp.stack([lp['lin_b'] for lp in p['layers']])
    wp, bp = _prep(p['edge_W'], p['edge_b'].reshape(1, HID), lin_Ws, lin_bs)

    xp = jnp.pad(x, ((0, 0), (0, 16 - 9)))
    node_Wp = jnp.pad(p['node_W'], ((0, 16 - 9), (0, 0)))
    h, ht = _encode(xp, node_Wp, p['node_b'].reshape(1, HID), bp[0:1])

    pad_e = E_PAD - E
    pad_cols = jnp.stack([jnp.zeros((pad_e,), jnp.int32),
                          jnp.full((pad_e,), DUMMY_DST, jnp.int32)])
    eidx_flat = jnp.concatenate([edge_index, pad_cols], axis=1)
    eidx = jnp.stack([eidx_flat[0].reshape(NBLK, EB),
                      eidx_flat[1].reshape(NBLK, EB)],
                     axis=1).reshape(NBLK * 2, EB)
    attr_p = jnp.pad(edge_attr, ((0, pad_e), (0, 0))).reshape(-1)
    batch3 = batch.reshape(NRB, 1, RB)

    for l in range(NUM_LAYERS):
        lp = p['layers'][l]
        aggs = _sc_layer(ht, eidx, attr_p, wp[l])
        args = (h, aggs[0], aggs[1],
                lp['W1'], lp['b1'].reshape(1, 2 * HID),
                lp['W2'], lp['b2'].reshape(1, HID),
                lp['ln_g'].reshape(1, HID), lp['ln_b'].reshape(1, HID))
        if l < NUM_LAYERS - 1:
            h, ht = _mlp(*args, bp[l + 1:l + 2])
        else:
            out = _mlp_final(*args, batch3)
    return out


# speedup vs baseline: 4.4786x; 1.0051x over previous
import functools

import jax
import jax.numpy as jnp
from jax import lax
from jax.experimental import pallas as pl
from jax.experimental.pallas import tpu as pltpu
from jax.experimental.pallas import tpu_sc as plsc

N = 10000
E = 640000
HID = 128
NUM_LAYERS = 4
EDGE_DIM = 4
NUM_GRAPHS = 64
LN_EPS = 1e-5

NC = 2
NS = 16
LANES = 16
NW = NC * NS

EB = 128
BLK_W = 157
NBLK = NW * BLK_W
E_PAD = NBLK * EB
N_PAD = 10112
ROWS_PER_SUB = N_PAD // NS
DUMMY_DST = N_PAD - 1
NCHUNK = HID // LANES

RB = 1000
NRB = N // RB


def _sc_msg_body(ht_hbm, eidx_hbm, attr_hbm, wp_hbm, out_hbm,
                 agg_sh, eidx0_v, eidx1_v, attr0_v, attr1_v,
                 rows0_v, rows1_v, wp_v,
                 gsem0, gsem1, ssem0, ssem1,
                 isem0, isem1, asem0, asem1):
    gsem = (gsem0, gsem1)
    ssem = (ssem0, ssem1)
    isem = (isem0, isem1)
    asem = (asem0, asem1)
    core = lax.axis_index("c")
    sub = lax.axis_index("s")
    wid = sub * NC + core

    eidx_v = (eidx0_v, eidx1_v)
    attr_v = (attr0_v, attr1_v)
    rows_v = (rows0_v, rows1_v)

    zvec = jnp.zeros((LANES,), jnp.float32)

    def zero_row(r, carry):
        for c in range(NCHUNK):
            rows0_v[r, pl.ds(c * LANES, LANES)] = zvec
        return carry

    lax.fori_loop(0, EB, zero_row, 0)
    for j in range(ROWS_PER_SUB // EB):
        pltpu.sync_copy(
            rows0_v,
            agg_sh.at[pl.ds(sub * ROWS_PER_SUB + j * EB, EB)])
    rem = ROWS_PER_SUB - (ROWS_PER_SUB // EB) * EB
    if rem:
        pltpu.sync_copy(
            rows0_v.at[pl.ds(0, rem)],
            agg_sh.at[pl.ds(sub * ROWS_PER_SUB + (ROWS_PER_SUB // EB) * EB,
                            rem)])

    pltpu.sync_copy(wp_hbm, wp_v)
    wvec = [[wp_v[k, pl.ds(c * LANES, LANES)] for c in range(NCHUNK)]
            for k in range(EDGE_DIM)]

    plsc.subcore_barrier()

    def issue_idx(j, b):
        blk = wid * BLK_W + j
        pltpu.async_copy(eidx_hbm.at[pl.ds(blk * 2, 2)], eidx_v[b], isem[b])
        pltpu.async_copy(
            attr_hbm.at[pl.ds(blk * EB * EDGE_DIM, EB * EDGE_DIM)],
            attr_v[b], asem[b])

    def wait_eidx(j, b):
        blk = wid * BLK_W + j
        pltpu.make_async_copy(eidx_hbm.at[pl.ds(blk * 2, 2)], eidx_v[b],
                              isem[b]).wait()

    def wait_attr(j, b):
        blk = wid * BLK_W + j
        pltpu.make_async_copy(
            attr_hbm.at[pl.ds(blk * EB * EDGE_DIM, EB * EDGE_DIM)],
            attr_v[b], asem[b]).wait()

    def issue_gather(b):
        pltpu.async_copy(ht_hbm.at[eidx_v[b].at[0]], rows_v[b], gsem[b])

    def wait_gather(b):
        pltpu.make_async_copy(
            ht_hbm.at[eidx_v[b].at[0]], rows_v[b], gsem[b]).wait()

    def issue_scatter(b):
        pltpu.async_copy(rows_v[b], agg_sh.at[eidx_v[b].at[1]], ssem[b],
                         add=True)

    def wait_scatter(b):
        pltpu.make_async_copy(
            rows_v[b], agg_sh.at[eidx_v[b].at[1]], ssem[b]).wait()

    def lane_bcast(vec, lane):
        return lax.gather(
            vec, jnp.full((LANES, 1), lane, jnp.int32),
            lax.GatherDimensionNumbers((), (0,), (0,)), (1,),
            mode=lax.GatherScatterMode.PROMISE_IN_BOUNDS)

    def compute(b):
        rv = rows_v[b]
        av = attr_v[b]

        def do_grp(q, ecarry):
            av16 = av[pl.ds(q * (4 * EDGE_DIM), LANES)]
            for t in range(4):
                e = q * 4 + t
                a = [lane_bcast(av16, t * EDGE_DIM + k)
                     for k in range(EDGE_DIM)]
                for c in range(NCHUNK):
                    acc = rv[e, pl.ds(c * LANES, LANES)]
                    for k in range(EDGE_DIM):
                        acc = acc + a[k] * wvec[k][c]
                    rv[e, pl.ds(c * LANES, LANES)] = jnp.maximum(acc, 0.0)
            return ecarry

        lax.fori_loop(0, EB // 4, do_grp, 0)

    def step(j, jm, first=False, has1=True):
        br = jm % 2
        if not first:
            wait_scatter(1 - br)
        if has1:
            issue_idx(j + 1, 1 - br)
            wait_eidx(j + 1, 1 - br)
            issue_gather(1 - br)
        wait_gather(br)
        wait_attr(j, br)
        compute(br)
        issue_scatter(br)

    issue_idx(0, 0)
    wait_eidx(0, 0)
    issue_gather(0)
    step(0, 0, first=True)

    def pair(p, carry):
        j = 1 + 2 * p
        step(j, 1)
        step(j + 1, 0)
        return carry

    lax.fori_loop(0, (BLK_W - 3) // 2, pair, 0)
    step(BLK_W - 2, 1)
    step(BLK_W - 1, 0, has1=False)
    wait_scatter((BLK_W - 1) % 2)

    plsc.subcore_barrier()

    pltpu.sync_copy(
        agg_sh.at[pl.ds(sub * ROWS_PER_SUB, ROWS_PER_SUB)],
        out_hbm.at[core, pl.ds(sub * ROWS_PER_SUB, ROWS_PER_SUB)])


_sc_layer = pl.kernel(
    _sc_msg_body,
    out_type=jax.ShapeDtypeStruct((NC, N_PAD, HID), jnp.float32),
    mesh=plsc.VectorSubcoreMesh(core_axis_name="c", subcore_axis_name="s"),
    scratch_types=[
        pltpu.VMEM_SHARED((N_PAD, HID), jnp.float32),
        pltpu.VMEM((2, EB), jnp.int32),
        pltpu.VMEM((2, EB), jnp.int32),
        pltpu.VMEM((EB * EDGE_DIM,), jnp.float32),
        pltpu.VMEM((EB * EDGE_DIM,), jnp.float32),
        pltpu.VMEM((EB, HID), jnp.float32),
        pltpu.VMEM((EB, HID), jnp.float32),
        pltpu.VMEM((EDGE_DIM, HID), jnp.float32),
        pltpu.SemaphoreType.DMA,
        pltpu.SemaphoreType.DMA,
        pltpu.SemaphoreType.DMA,
        pltpu.SemaphoreType.DMA,
        pltpu.SemaphoreType.DMA,
        pltpu.SemaphoreType.DMA,
        pltpu.SemaphoreType.DMA,
        pltpu.SemaphoreType.DMA,
    ],
    compiler_params=pltpu.CompilerParams(needs_layout_passes=False),
)


def _prep_body(edge_W_ref, edge_b_ref, lin_Ws_ref, lin_bs_ref,
               wp_ref, bp_ref):
    ew = edge_W_ref[...]
    eb = edge_b_ref[...]
    for l in range(NUM_LAYERS):
        lw = lin_Ws_ref[l]
        wp_ref[l] = jnp.dot(ew, lw, preferred_element_type=jnp.float32)
        bp_ref[pl.ds(l, 1), :] = (
            jnp.dot(eb, lw, preferred_element_type=jnp.float32)
            + lin_bs_ref[pl.ds(l, 1), :])


def _prep(edge_W, edge_b, lin_Ws, lin_bs):
    return pl.pallas_call(
        _prep_body,
        out_shape=(
            jax.ShapeDtypeStruct((NUM_LAYERS, EDGE_DIM, HID), jnp.float32),
            jax.ShapeDtypeStruct((NUM_LAYERS, HID), jnp.float32),
        ),
    )(edge_W, edge_b, lin_Ws, lin_bs)


def _encode_body(x_ref, w_ref, b_ref, bp0_ref, h_ref, ht_ref):
    h = (jnp.dot(x_ref[...], w_ref[...], preferred_element_type=jnp.float32)
         + b_ref[...])
    h_ref[...] = h
    ht_ref[...] = h + bp0_ref[...]


def _encode(xp, node_Wp, node_b, bp0):
    return pl.pallas_call(
        _encode_body,
        grid=(NRB,),
        in_specs=[
            pl.BlockSpec((RB, 16), lambda i: (i, 0)),
            pl.BlockSpec((16, HID), lambda i: (0, 0)),
            pl.BlockSpec((1, HID), lambda i: (0, 0)),
            pl.BlockSpec((1, HID), lambda i: (0, 0)),
        ],
        out_specs=(
            pl.BlockSpec((RB, HID), lambda i: (i, 0)),
            pl.BlockSpec((RB, HID), lambda i: (i, 0)),
        ),
        out_shape=(
            jax.ShapeDtypeStruct((N, HID), jnp.float32),
            jax.ShapeDtypeStruct((N, HID), jnp.float32),
        ),
    )(xp, node_Wp, node_b, bp0)


def _mlp_core(h, agg0, agg1, w1, b1, w2, b2, g, bln):
    z = h + agg0 + agg1
    u = jnp.maximum(
        jnp.dot(z, w1, preferred_element_type=jnp.float32) + b1, 0.0)
    v = jnp.dot(u, w2, preferred_element_type=jnp.float32) + b2
    m = jnp.mean(v, axis=1, keepdims=True)
    d = v - m
    var = jnp.mean(d * d, axis=1, keepdims=True)
    ln = d * lax.rsqrt(var + LN_EPS) * g + bln
    return jnp.maximum(ln, 0.0) + h


def _mlp_body(h_ref, a0_ref, a1_ref, w1_ref, b1_ref, w2_ref, b2_ref,
              g_ref, bln_ref, bpn_ref, h_out_ref, ht_out_ref):
    hn = _mlp_core(h_ref[...], a0_ref[...], a1_ref[...], w1_ref[...],
                   b1_ref[...], w2_ref[...], b2_ref[...], g_ref[...],
                   bln_ref[...])
    h_out_ref[...] = hn
    ht_out_ref[...] = hn + bpn_ref[...]


def _mlp(h, agg0, agg1, w1, b1, w2, b2, g, bln, bpn):
    full = lambda shape: pl.BlockSpec(shape, lambda i: tuple(0 for _ in shape))
    row = pl.BlockSpec((RB, HID), lambda i: (i, 0))
    return pl.pallas_call(
        _mlp_body,
        grid=(NRB,),
        in_specs=[
            row, row, row,
            full((HID, 2 * HID)), full((1, 2 * HID)),
            full((2 * HID, HID)), full((1, HID)),
            full((1, HID)), full((1, HID)), full((1, HID)),
        ],
        out_specs=(row, row),
        out_shape=(
            jax.ShapeDtypeStruct((N, HID), jnp.float32),
            jax.ShapeDtypeStruct((N, HID), jnp.float32),
        ),
    )(h, agg0, agg1, w1, b1, w2, b2, g, bln, bpn)


def _mlp_final_body(h_ref, a0_ref, a1_ref, w1_ref, b1_ref, w2_ref, b2_ref,
                    g_ref, bln_ref, batch_ref, out_ref):
    hn = _mlp_core(h_ref[...], a0_ref[...], a1_ref[...], w1_ref[...],
                   b1_ref[...], w2_ref[...], b2_ref[...], g_ref[...],
                   bln_ref[...])

    @pl.when(pl.program_id(0) == 0)
    def _():
        out_ref[...] = jnp.zeros_like(out_ref)

    bb = batch_ref[0]
    oh = (lax.broadcasted_iota(jnp.int32, (NUM_GRAPHS, RB), 0)
          == bb).astype(jnp.float32)
    out_ref[...] += jnp.dot(oh, hn, preferred_element_type=jnp.float32)


def _mlp_final(h, agg0, agg1, w1, b1, w2, b2, g, bln, batch3):
    full = lambda shape: pl.BlockSpec(shape, lambda i: tuple(0 for _ in shape))
    row = pl.BlockSpec((RB, HID), lambda i: (i, 0))
    return pl.pallas_call(
        _mlp_final_body,
        grid=(NRB,),
        in_specs=[
            row, row, row,
            full((HID, 2 * HID)), full((1, 2 * HID)),
            full((2 * HID, HID)), full((1, HID)),
            full((1, HID)), full((1, HID)),
            pl.BlockSpec((1, 1, RB), lambda i: (i, 0, 0)),
        ],
        out_specs=pl.BlockSpec((NUM_GRAPHS, HID), lambda i: (0, 0)),
        out_shape=jax.ShapeDtypeStruct((NUM_GRAPHS, HID), jnp.float32),
    )(h, agg0, agg1, w1, b1, w2, b2, g, bln, batch3)


def kernel(x, edge_index, edge_attr, batch, params):
    p = params
    lin_Ws = jnp.stack([lp['lin_W'] for lp in p['layers']])
    lin_bs = jnp.stack([lp['lin_b'] for lp in p['layers']])
    wp, bp = _prep(p['edge_W'], p['edge_b'].reshape(1, HID), lin_Ws, lin_bs)

    xp = jnp.pad(x, ((0, 0), (0, 16 - 9)))
    node_Wp = jnp.pad(p['node_W'], ((0, 16 - 9), (0, 0)))
    h, ht = _encode(xp, node_Wp, p['node_b'].reshape(1, HID), bp[0:1])

    pad_e = E_PAD - E
    pad_cols = jnp.stack([jnp.zeros((pad_e,), jnp.int32),
                          jnp.full((pad_e,), DUMMY_DST, jnp.int32)])
    eidx_flat = jnp.concatenate([edge_index, pad_cols], axis=1)
    eidx = jnp.stack([eidx_flat[0].reshape(NBLK, EB),
                      eidx_flat[1].reshape(NBLK, EB)],
                     axis=1).reshape(NBLK * 2, EB)
    attr_p = jnp.pad(edge_attr, ((0, pad_e), (0, 0))).reshape(-1)
    batch3 = batch.reshape(NRB, 1, RB)

    for l in range(NUM_LAYERS):
        lp = p['layers'][l]
        aggs = _sc_layer(ht, eidx, attr_p, wp[l])
        args = (h, aggs[0], aggs[1],
                lp['W1'], lp['b1'].reshape(1, 2 * HID),
                lp['W2'], lp['b2'].reshape(1, HID),
                lp['ln_g'].reshape(1, HID), lp['ln_b'].reshape(1, HID))
        if l < NUM_LAYERS - 1:
            h, ht = _mlp(*args, bp[l + 1:l + 2])
        else:
            out = _mlp_final(*args, batch3)
    return out
